# Initial kernel scaffold; baseline (speedup 1.0000x reference)
import jax
import jax.numpy as jnp
from jax.experimental import pallas as pl


def kernel(query, base_memory, lora_A, lora_B, gru_w_ih, gru_w_hh, gru_b_ih, gru_b_hh, write_w, write_b, erase_w, erase_b):
    raise NotImplementedError("write your pallas kernel here")


# speedup vs baseline: 23.6588x; 23.6588x over previous
import functools

import jax
import jax.numpy as jnp
from jax import lax
from jax.experimental import pallas as pl
from jax.experimental.pallas import tpu as pltpu
from jax.experimental.pallas import tpu_sc as plsc

_N = 100000
_D = 128
_R = 16
_K = 8
_B = 1024
_CBLK = 2000
_NBLK = _N // _CBLK
_U = _B * _K
_RB = 256
_CJ = 128
_NJ = _U // _CJ
_BIG = 2**30


def _qmean(query):
    lq = query.shape[1]

    def body(q_ref, o_ref):
        acc = q_ref[:, 0, :]
        for l in range(1, lq):
            acc = acc + q_ref[:, l, :]
        o_ref[...] = acc * (1.0 / lq)

    return pl.pallas_call(
        body,
        grid=(4,),
        in_specs=[pl.BlockSpec((_B // 4, lq, _D), lambda g: (g, 0, 0))],
        out_specs=pl.BlockSpec((_B // 4, _D), lambda g: (g, 0)),
        out_shape=jax.ShapeDtypeStruct((_B, _D), jnp.float32),
    )(query)


def _score_topk(q, base, lora_a, lora_b):
    def body(q_ref, b_ref, la_ref, lb_ref, mem_ref, idx_ref, bv_s, bi_s):
        g = pl.program_id(0)
        mem_blk = b_ref[...] + jnp.dot(la_ref[...], lb_ref[...],
                                       preferred_element_type=jnp.float32)
        mem_ref[...] = mem_blk
        scores = lax.dot_general(q_ref[...], mem_blk, (((1,), (1,)), ((), ())),
                                 preferred_element_type=jnp.float32)
        col = lax.broadcasted_iota(jnp.int32, (_B, _CBLK), 1) + g * _CBLK

        @pl.when(g == 0)
        def _():
            bv_s[...] = jnp.full((_B, _K), -jnp.inf, jnp.float32)
            bi_s[...] = jnp.full((_B, _K), _BIG, jnp.int32)

        s = scores
        bv_list, bi_list = [], []
        for _ in range(_K):
            m = jnp.max(s, axis=1, keepdims=True)
            isel = jnp.min(jnp.where(s == m, col, _BIG), axis=1, keepdims=True)
            bv_list.append(m)
            bi_list.append(isel)
            s = jnp.where(col == isel, -jnp.inf, s)
        cv = jnp.concatenate([bv_s[...]] + bv_list, axis=1)
        ci = jnp.concatenate([bi_s[...]] + bi_list, axis=1)
        nv, ni = [], []
        for _ in range(_K):
            m = jnp.max(cv, axis=1, keepdims=True)
            isel = jnp.min(jnp.where(cv == m, ci, _BIG), axis=1, keepdims=True)
            nv.append(m)
            ni.append(isel)
            cv = jnp.where(ci == isel, -jnp.inf, cv)
        bv_s[...] = jnp.concatenate(nv, axis=1)
        bi_s[...] = jnp.concatenate(ni, axis=1)

        @pl.when(g == _NBLK - 1)
        def _():
            idx_ref[...] = bi_s[...]

    return pl.pallas_call(
        body,
        grid=(_NBLK,),
        in_specs=[
            pl.BlockSpec((_B, _D), lambda g: (0, 0)),
            pl.BlockSpec((_CBLK, _D), lambda g: (g, 0)),
            pl.BlockSpec((_CBLK, _R), lambda g: (g, 0)),
            pl.BlockSpec((_R, _D), lambda g: (0, 0)),
        ],
        out_specs=[
            pl.BlockSpec((_CBLK, _D), lambda g: (g, 0)),
            pl.BlockSpec((_B, _K), lambda g: (0, 0)),
        ],
        out_shape=[
            jax.ShapeDtypeStruct((_N, _D), jnp.float32),
            jax.ShapeDtypeStruct((_B, _K), jnp.int32),
        ],
        scratch_shapes=[pltpu.VMEM((_B, _K), jnp.float32),
                        pltpu.VMEM((_B, _K), jnp.int32)],
    )(q, base, lora_a, lora_b)


def _sc_gather(mem, base, idx2d):
    mesh = plsc.VectorSubcoreMesh(core_axis_name="c", subcore_axis_name="s")
    nrows = idx2d.shape[0]
    rpw = nrows // 32
    ipw = rpw * 128

    @functools.partial(
        pl.kernel,
        mesh=mesh,
        out_type=(jax.ShapeDtypeStruct((_U, _D), jnp.float32),
                  jax.ShapeDtypeStruct((_U, _D), jnp.float32)),
        scratch_types=[pltpu.VMEM((rpw, 128), jnp.int32),
                       pltpu.VMEM((ipw, _D), jnp.float32),
                       pltpu.VMEM((ipw, _D), jnp.float32),
                       pltpu.SemaphoreType.DMA],
    )
    def k(mem_hbm, base_hbm, idx_hbm, ret_out, br_out, idx_v, rows_v, rows2_v, sem):
        wid = lax.axis_index("s") * 2 + lax.axis_index("c")
        pltpu.sync_copy(idx_hbm.at[pl.ds(wid * rpw, rpw)], idx_v)
        for c in range(rpw):
            pltpu.async_copy(mem_hbm.at[idx_v.at[c]],
                             rows_v.at[pl.ds(c * 128, 128)], sem).wait()
            pltpu.async_copy(base_hbm.at[idx_v.at[c]],
                             rows2_v.at[pl.ds(c * 128, 128)], sem).wait()
        pltpu.sync_copy(rows_v, ret_out.at[pl.ds(wid * ipw, ipw)])
        pltpu.sync_copy(rows2_v, br_out.at[pl.ds(wid * ipw, ipw)])

    return k(mem, base, idx2d)


def _gru(ret3, w_ih, b_ih2, b_hh2, w_w, w_b2):
    def body(r_ref, wih_ref, bih_ref, bhh_ref, ww_ref, wb_ref,
             h_ref, w_ref, sp_ref):
        x = r_ref[:, 0, :]
        for kk in range(1, _K):
            x = x + r_ref[:, kk, :]
        gi = lax.dot_general(x, wih_ref[...], (((1,), (1,)), ((), ())),
                             preferred_element_type=jnp.float32) + bih_ref[...]
        gh = bhh_ref[...]
        i_r = gi[:, :_D]
        i_z = gi[:, _D:2 * _D]
        i_n = gi[:, 2 * _D:]
        h_r = gh[:, :_D]
        h_z = gh[:, _D:2 * _D]
        h_n = gh[:, 2 * _D:]
        r = jax.nn.sigmoid(i_r + h_r)
        z = jax.nn.sigmoid(i_z + h_z)
        n = jnp.tanh(i_n + r * h_n)
        hidden = (1.0 - z) * n
        h_ref[...] = hidden
        wl = jnp.sum(hidden * ww_ref[...], axis=1, keepdims=True) + wb_ref[0, 0]
        w_ref[...] = jnp.broadcast_to(jax.nn.sigmoid(wl), (_B, _D))
        sp = jnp.maximum(wl, 0.0) + jnp.log1p(jnp.exp(-jnp.abs(wl)))
        sp_ref[...] = jnp.broadcast_to(sp, (_B, _D))

    return pl.pallas_call(
        body,
        in_specs=[
            pl.BlockSpec(memory_space=pltpu.VMEM),
            pl.BlockSpec(memory_space=pltpu.VMEM),
            pl.BlockSpec(memory_space=pltpu.VMEM),
            pl.BlockSpec(memory_space=pltpu.VMEM),
            pl.BlockSpec(memory_space=pltpu.VMEM),
            pl.BlockSpec(memory_space=pltpu.SMEM),
        ],
        out_shape=[
            jax.ShapeDtypeStruct((_B, _D), jnp.float32),
            jax.ShapeDtypeStruct((_B, _D), jnp.float32),
            jax.ShapeDtypeStruct((_B, _D), jnp.float32),
        ],
    )(ret3, w_ih, b_ih2, b_hh2, w_w, w_b2)


def _write_sums(idxr_w, idxc, spc):
    def body(idxr_ref, idxc_ref, spc_ref, tot_ref, lat_ref):
        g = pl.program_id(0)
        ir = idxr_ref[...]
        tot = jnp.zeros((_RB, 1), jnp.float32)
        lat = jnp.zeros((_RB, 1), jnp.float32)
        rowg = lax.broadcasted_iota(jnp.int32, (_RB, _CJ), 0) + g * _RB
        for c in range(_NJ):
            ic = idxc_ref[:, c * _CJ:(c + 1) * _CJ]
            spv = spc_ref[:, c * _CJ:(c + 1) * _CJ]
            eq = ir == ic
            tot = tot + jnp.sum(jnp.where(eq, spv, 0.0), axis=1, keepdims=True)
            colg = lax.broadcasted_iota(jnp.int32, (_RB, _CJ), 1) + c * _CJ
            lat = lat + jnp.sum(jnp.where(eq & (colg > rowg), spv, 0.0),
                                axis=1, keepdims=True)
        tot_ref[...] = jnp.broadcast_to(tot, (_RB, _D))
        lat_ref[...] = jnp.broadcast_to(lat, (_RB, _D))

    return pl.pallas_call(
        body,
        grid=(_U // _RB,),
        in_specs=[
            pl.BlockSpec((_RB, _D), lambda g: (g, 0)),
            pl.BlockSpec((1, _U), lambda g: (0, 0)),
            pl.BlockSpec((1, _U), lambda g: (0, 0)),
        ],
        out_specs=[
            pl.BlockSpec((_RB, _D), lambda g: (g, 0)),
            pl.BlockSpec((_RB, _D), lambda g: (g, 0)),
        ],
        out_shape=[
            jax.ShapeDtypeStruct((_U, _D), jnp.float32),
            jax.ShapeDtypeStruct((_U, _D), jnp.float32),
        ],
    )(idxr_w, idxc, spc)


def _write_rows(idxr_w, idxc, latc, wc, v_u, base_rows, tot_w):
    def body(idxr_ref, idxc_ref, latc_ref, wc_ref, v_ref, br_ref, tot_ref,
             out_ref):
        ir = idxr_ref[...]
        coef = wc_ref[...] * jnp.exp(-latc_ref[...])
        acc = jnp.zeros((_RB, _D), jnp.float32)
        for c in range(_NJ):
            ic = idxc_ref[:, c * _CJ:(c + 1) * _CJ]
            m = jnp.where(ir == ic, coef[:, c * _CJ:(c + 1) * _CJ], 0.0)
            acc = acc + jnp.dot(m, v_ref[c * _CJ:(c + 1) * _CJ, :],
                                preferred_element_type=jnp.float32)
        out_ref[...] = jnp.exp(-tot_ref[...]) * br_ref[...] + acc

    return pl.pallas_call(
        body,
        grid=(_U // _RB,),
        in_specs=[
            pl.BlockSpec((_RB, _D), lambda g: (g, 0)),
            pl.BlockSpec((1, _U), lambda g: (0, 0)),
            pl.BlockSpec((1, _U), lambda g: (0, 0)),
            pl.BlockSpec((1, _U), lambda g: (0, 0)),
            pl.BlockSpec((_U, _D), lambda g: (0, 0)),
            pl.BlockSpec((_RB, _D), lambda g: (g, 0)),
            pl.BlockSpec((_RB, _D), lambda g: (g, 0)),
        ],
        out_specs=pl.BlockSpec((_RB, _D), lambda g: (g, 0)),
        out_shape=jax.ShapeDtypeStruct((_U, _D), jnp.float32),
    )(idxr_w, idxc, latc, wc, v_u, base_rows, tot_w)


def _sc_scatter(base, idx2d, rows):
    mesh = plsc.VectorSubcoreMesh(core_axis_name="c", subcore_axis_name="s")
    nrows = idx2d.shape[0]
    rpw = nrows // 16
    ipw = rpw * 128

    @functools.partial(
        pl.kernel,
        mesh=mesh,
        out_type=jax.ShapeDtypeStruct((_N, _D), jnp.float32),
        scratch_types=[pltpu.VMEM((rpw, 128), jnp.int32),
                       pltpu.VMEM((ipw, _D), jnp.float32),
                       pltpu.SemaphoreType.DMA],
    )
    def k(base_hbm, idx_hbm, rows_hbm, out_hbm, idx_v, rows_v, sem):
        cid = lax.axis_index("c")
        sid = lax.axis_index("s")

        @pl.when(cid == 0)
        def _():
            for j in range(7):
                ch = sid + 16 * j

                @pl.when(ch < 100)
                def _():
                    pltpu.sync_copy(base_hbm.at[pl.ds(ch * 1000, 1000)],
                                    out_hbm.at[pl.ds(ch * 1000, 1000)])
            plsc.subcore_barrier()
            pltpu.sync_copy(idx_hbm.at[pl.ds(sid * rpw, rpw)], idx_v)
            pltpu.sync_copy(rows_hbm.at[pl.ds(sid * ipw, ipw)], rows_v)
            for c in range(rpw):
                pltpu.async_copy(rows_v.at[pl.ds(c * 128, 128)],
                                 out_hbm.at[idx_v.at[c]], sem).wait()

    return k(base, idx2d, rows)


def kernel(query, base_memory, lora_A, lora_B, gru_w_ih, gru_w_hh, gru_b_ih,
           gru_b_hh, write_w, write_b, erase_w, erase_b):
    q = _qmean(query)
    mem, top_idx = _score_topk(q, base_memory, lora_A, lora_B)
    idx2d = top_idx.reshape(_U // 128, 128)
    retrieved_flat, base_rows = _sc_gather(mem, base_memory, idx2d)
    hidden, w128, sp128 = _gru(retrieved_flat.reshape(_B, _K, _D), gru_w_ih,
                               gru_b_ih.reshape(1, -1), gru_b_hh.reshape(1, -1),
                               write_w, write_b.reshape(1, 1))
    w = w128[:, :1]
    sp = sp128[:, :1]
    idxf = top_idx.reshape(-1).astype(jnp.float32)
    idxr_w = jnp.broadcast_to(idxf[:, None], (_U, _D))
    idxc = idxf.reshape(1, _U)
    spc = jnp.broadcast_to(sp, (_B, _K)).reshape(1, _U)
    wc = jnp.broadcast_to(w, (_B, _K)).reshape(1, _U)
    tot_w, lat_w = _write_sums(idxr_w, idxc, spc)
    v_u = jnp.broadcast_to(q[:, None, :], (_B, _K, _D)).reshape(_U, _D)
    rows = _write_rows(idxr_w, idxc, lat_w[:, :1].reshape(1, _U), wc, v_u,
                       base_rows, tot_w)
    new_memory = _sc_scatter(base_memory, idx2d, rows)
    return (retrieved_flat.reshape(_B, _K, _D), hidden, new_memory)


# speedup vs baseline: 50.6091x; 2.1391x over previous
import functools

import jax
import jax.numpy as jnp
from jax import lax
from jax.experimental import pallas as pl
from jax.experimental.pallas import tpu as pltpu
from jax.experimental.pallas import tpu_sc as plsc

_N = 100000
_D = 128
_R = 16
_K = 8
_B = 1024
_CBLK = 2000
_NBLK = _N // _CBLK
_U = _B * _K
_RB = 256
_CJ = 128
_NJ = _U // _CJ
_BIG = 2**30


def _qmean(query):
    lq = query.shape[1]

    def body(q_ref, o_ref):
        acc = q_ref[:, 0, :]
        for l in range(1, lq):
            acc = acc + q_ref[:, l, :]
        o_ref[...] = acc * (1.0 / lq)

    return pl.pallas_call(
        body,
        grid=(4,),
        in_specs=[pl.BlockSpec((_B // 4, lq, _D), lambda g: (g, 0, 0))],
        out_specs=pl.BlockSpec((_B // 4, _D), lambda g: (g, 0)),
        out_shape=jax.ShapeDtypeStruct((_B, _D), jnp.float32),
    )(query)


def _score_topk(q, base, lora_a, lora_b):
    def body(q_ref, b_ref, la_ref, lb_ref, mem_ref, idx_ref, cp_ref, bv_s, bi_s):
        g = pl.program_id(0)
        base_blk = b_ref[...]
        cp_ref[...] = base_blk
        mem_blk = base_blk + jnp.dot(la_ref[...], lb_ref[...],
                                     preferred_element_type=jnp.float32)
        mem_ref[...] = mem_blk
        scores = lax.dot_general(q_ref[...], mem_blk, (((1,), (1,)), ((), ())),
                                 preferred_element_type=jnp.float32)
        col = lax.broadcasted_iota(jnp.int32, (_B, _CBLK), 1) + g * _CBLK

        @pl.when(g == 0)
        def _():
            bv_s[...] = jnp.full((_B, _K), -jnp.inf, jnp.float32)
            bi_s[...] = jnp.full((_B, _K), _BIG, jnp.int32)

        s = scores
        bv_list, bi_list = [], []
        for _ in range(_K):
            m = jnp.max(s, axis=1, keepdims=True)
            isel = jnp.min(jnp.where(s == m, col, _BIG), axis=1, keepdims=True)
            bv_list.append(m)
            bi_list.append(isel)
            s = jnp.where(col == isel, -jnp.inf, s)
        cv = jnp.concatenate([bv_s[...]] + bv_list, axis=1)
        ci = jnp.concatenate([bi_s[...]] + bi_list, axis=1)
        nv, ni = [], []
        for _ in range(_K):
            m = jnp.max(cv, axis=1, keepdims=True)
            isel = jnp.min(jnp.where(cv == m, ci, _BIG), axis=1, keepdims=True)
            nv.append(m)
            ni.append(isel)
            cv = jnp.where(ci == isel, -jnp.inf, cv)
        bv_s[...] = jnp.concatenate(nv, axis=1)
        bi_s[...] = jnp.concatenate(ni, axis=1)

        @pl.when(g == _NBLK - 1)
        def _():
            idx_ref[...] = bi_s[...]

    return pl.pallas_call(
        body,
        grid=(_NBLK,),
        in_specs=[
            pl.BlockSpec((_B, _D), lambda g: (0, 0)),
            pl.BlockSpec((_CBLK, _D), lambda g: (g, 0)),
            pl.BlockSpec((_CBLK, _R), lambda g: (g, 0)),
            pl.BlockSpec((_R, _D), lambda g: (0, 0)),
        ],
        out_specs=[
            pl.BlockSpec((_CBLK, _D), lambda g: (g, 0)),
            pl.BlockSpec((_B, _K), lambda g: (0, 0)),
            pl.BlockSpec((_CBLK, _D), lambda g: (g, 0)),
        ],
        out_shape=[
            jax.ShapeDtypeStruct((_N, _D), jnp.float32),
            jax.ShapeDtypeStruct((_B, _K), jnp.int32),
            jax.ShapeDtypeStruct((_N, _D), jnp.float32),
        ],
        scratch_shapes=[pltpu.VMEM((_B, _K), jnp.float32),
                        pltpu.VMEM((_B, _K), jnp.int32)],
    )(q, base, lora_a, lora_b)


def _sc_gather(mem, base, idx2d):
    mesh = plsc.VectorSubcoreMesh(core_axis_name="c", subcore_axis_name="s")
    nrows = idx2d.shape[0]
    rpw = nrows // 32
    ipw = rpw * 128

    @functools.partial(
        pl.kernel,
        mesh=mesh,
        out_type=(jax.ShapeDtypeStruct((_U, _D), jnp.float32),
                  jax.ShapeDtypeStruct((_U, _D), jnp.float32)),
        scratch_types=[pltpu.VMEM((rpw, 128), jnp.int32),
                       pltpu.VMEM((ipw, _D), jnp.float32),
                       pltpu.VMEM((ipw, _D), jnp.float32),
                       pltpu.SemaphoreType.DMA],
    )
    def k(mem_hbm, base_hbm, idx_hbm, ret_out, br_out, idx_v, rows_v, rows2_v, sem):
        wid = lax.axis_index("s") * 2 + lax.axis_index("c")
        pltpu.sync_copy(idx_hbm.at[pl.ds(wid * rpw, rpw)], idx_v)
        for c in range(rpw):
            pltpu.async_copy(mem_hbm.at[idx_v.at[c]],
                             rows_v.at[pl.ds(c * 128, 128)], sem).wait()
            pltpu.async_copy(base_hbm.at[idx_v.at[c]],
                             rows2_v.at[pl.ds(c * 128, 128)], sem).wait()
        pltpu.sync_copy(rows_v, ret_out.at[pl.ds(wid * ipw, ipw)])
        pltpu.sync_copy(rows2_v, br_out.at[pl.ds(wid * ipw, ipw)])

    return k(mem, base, idx2d)


def _gru(ret3, w_ih, b_ih2, b_hh2, w_w, w_b2):
    def body(r_ref, wih_ref, bih_ref, bhh_ref, ww_ref, wb_ref,
             h_ref, w_ref, sp_ref):
        x = r_ref[:, 0, :]
        for kk in range(1, _K):
            x = x + r_ref[:, kk, :]
        gi = lax.dot_general(x, wih_ref[...], (((1,), (1,)), ((), ())),
                             preferred_element_type=jnp.float32) + bih_ref[...]
        gh = bhh_ref[...]
        i_r = gi[:, :_D]
        i_z = gi[:, _D:2 * _D]
        i_n = gi[:, 2 * _D:]
        h_r = gh[:, :_D]
        h_z = gh[:, _D:2 * _D]
        h_n = gh[:, 2 * _D:]
        r = jax.nn.sigmoid(i_r + h_r)
        z = jax.nn.sigmoid(i_z + h_z)
        n = jnp.tanh(i_n + r * h_n)
        hidden = (1.0 - z) * n
        h_ref[...] = hidden
        wl = jnp.sum(hidden * ww_ref[...], axis=1, keepdims=True) + wb_ref[0, 0]
        w_ref[...] = jnp.broadcast_to(jax.nn.sigmoid(wl), (_B, _D))
        sp = jnp.maximum(wl, 0.0) + jnp.log1p(jnp.exp(-jnp.abs(wl)))
        sp_ref[...] = jnp.broadcast_to(sp, (_B, _D))

    return pl.pallas_call(
        body,
        in_specs=[
            pl.BlockSpec(memory_space=pltpu.VMEM),
            pl.BlockSpec(memory_space=pltpu.VMEM),
            pl.BlockSpec(memory_space=pltpu.VMEM),
            pl.BlockSpec(memory_space=pltpu.VMEM),
            pl.BlockSpec(memory_space=pltpu.VMEM),
            pl.BlockSpec(memory_space=pltpu.SMEM),
        ],
        out_shape=[
            jax.ShapeDtypeStruct((_B, _D), jnp.float32),
            jax.ShapeDtypeStruct((_B, _D), jnp.float32),
            jax.ShapeDtypeStruct((_B, _D), jnp.float32),
        ],
    )(ret3, w_ih, b_ih2, b_hh2, w_w, w_b2)


def _write_sums(idxr_w, idxc, spc):
    def body(idxr_ref, idxc_ref, spc_ref, tot_ref, lat_ref):
        g = pl.program_id(0)
        ir = idxr_ref[...]
        tot = jnp.zeros((_RB, 1), jnp.float32)
        lat = jnp.zeros((_RB, 1), jnp.float32)
        rowg = lax.broadcasted_iota(jnp.int32, (_RB, _CJ), 0) + g * _RB
        for c in range(_NJ):
            ic = idxc_ref[:, c * _CJ:(c + 1) * _CJ]
            spv = spc_ref[:, c * _CJ:(c + 1) * _CJ]
            eq = ir == ic
            tot = tot + jnp.sum(jnp.where(eq, spv, 0.0), axis=1, keepdims=True)
            colg = lax.broadcasted_iota(jnp.int32, (_RB, _CJ), 1) + c * _CJ
            lat = lat + jnp.sum(jnp.where(eq & (colg > rowg), spv, 0.0),
                                axis=1, keepdims=True)
        tot_ref[...] = jnp.broadcast_to(tot, (_RB, _D))
        lat_ref[...] = jnp.broadcast_to(lat, (_RB, _D))

    return pl.pallas_call(
        body,
        grid=(_U // _RB,),
        in_specs=[
            pl.BlockSpec((_RB, _D), lambda g: (g, 0)),
            pl.BlockSpec((1, _U), lambda g: (0, 0)),
            pl.BlockSpec((1, _U), lambda g: (0, 0)),
        ],
        out_specs=[
            pl.BlockSpec((_RB, _D), lambda g: (g, 0)),
            pl.BlockSpec((_RB, _D), lambda g: (g, 0)),
        ],
        out_shape=[
            jax.ShapeDtypeStruct((_U, _D), jnp.float32),
            jax.ShapeDtypeStruct((_U, _D), jnp.float32),
        ],
    )(idxr_w, idxc, spc)


def _write_rows(idxr_w, idxc, latc, wc, v_u, base_rows, tot_w):
    def body(idxr_ref, idxc_ref, latc_ref, wc_ref, v_ref, br_ref, tot_ref,
             out_ref):
        ir = idxr_ref[...]
        coef = wc_ref[...] * jnp.exp(-latc_ref[...])
        acc = jnp.zeros((_RB, _D), jnp.float32)
        for c in range(_NJ):
            ic = idxc_ref[:, c * _CJ:(c + 1) * _CJ]
            m = jnp.where(ir == ic, coef[:, c * _CJ:(c + 1) * _CJ], 0.0)
            acc = acc + jnp.dot(m, v_ref[c * _CJ:(c + 1) * _CJ, :],
                                preferred_element_type=jnp.float32)
        out_ref[...] = jnp.exp(-tot_ref[...]) * br_ref[...] + acc

    return pl.pallas_call(
        body,
        grid=(_U // _RB,),
        in_specs=[
            pl.BlockSpec((_RB, _D), lambda g: (g, 0)),
            pl.BlockSpec((1, _U), lambda g: (0, 0)),
            pl.BlockSpec((1, _U), lambda g: (0, 0)),
            pl.BlockSpec((1, _U), lambda g: (0, 0)),
            pl.BlockSpec((_U, _D), lambda g: (0, 0)),
            pl.BlockSpec((_RB, _D), lambda g: (g, 0)),
            pl.BlockSpec((_RB, _D), lambda g: (g, 0)),
        ],
        out_specs=pl.BlockSpec((_RB, _D), lambda g: (g, 0)),
        out_shape=jax.ShapeDtypeStruct((_U, _D), jnp.float32),
    )(idxr_w, idxc, latc, wc, v_u, base_rows, tot_w)


def _sc_scatter(newmem, idx2d, rows):
    mesh = plsc.VectorSubcoreMesh(core_axis_name="c", subcore_axis_name="s")
    nrows = idx2d.shape[0]
    rpw = nrows // 32
    ipw = rpw * 128

    @functools.partial(
        pl.kernel,
        mesh=mesh,
        out_type=(),
        scratch_types=[pltpu.VMEM((rpw, 128), jnp.int32),
                       pltpu.VMEM((ipw, _D), jnp.float32),
                       pltpu.SemaphoreType.DMA],
    )
    def k(idx_hbm, rows_hbm, out_hbm, idx_v, rows_v, sem):
        wid = lax.axis_index("s") * 2 + lax.axis_index("c")
        pltpu.sync_copy(idx_hbm.at[pl.ds(wid * rpw, rpw)], idx_v)
        pltpu.sync_copy(rows_hbm.at[pl.ds(wid * ipw, ipw)], rows_v)
        for c in range(rpw):
            pltpu.async_copy(rows_v.at[pl.ds(c * 128, 128)],
                             out_hbm.at[idx_v.at[c]], sem).wait()

    out_ref = jax.new_ref(newmem)
    k(idx2d, rows, out_ref)
    return out_ref[...]


def kernel(query, base_memory, lora_A, lora_B, gru_w_ih, gru_w_hh, gru_b_ih,
           gru_b_hh, write_w, write_b, erase_w, erase_b):
    q = _qmean(query)
    mem, top_idx, newmem = _score_topk(q, base_memory, lora_A, lora_B)
    idx2d = top_idx.reshape(_U // 128, 128)
    retrieved_flat, base_rows = _sc_gather(mem, base_memory, idx2d)
    hidden, w128, sp128 = _gru(retrieved_flat.reshape(_B, _K, _D), gru_w_ih,
                               gru_b_ih.reshape(1, -1), gru_b_hh.reshape(1, -1),
                               write_w, write_b.reshape(1, 1))
    w = w128[:, :1]
    sp = sp128[:, :1]
    idxf = top_idx.reshape(-1).astype(jnp.float32)
    idxr_w = jnp.broadcast_to(idxf[:, None], (_U, _D))
    idxc = idxf.reshape(1, _U)
    spc = jnp.broadcast_to(sp, (_B, _K)).reshape(1, _U)
    wc = jnp.broadcast_to(w, (_B, _K)).reshape(1, _U)
    tot_w, lat_w = _write_sums(idxr_w, idxc, spc)
    v_u = jnp.broadcast_to(q[:, None, :], (_B, _K, _D)).reshape(_U, _D)
    rows = _write_rows(idxr_w, idxc, lat_w[:, :1].reshape(1, _U), wc, v_u,
                       base_rows, tot_w)
    new_memory = _sc_scatter(newmem, idx2d, rows)
    return (retrieved_flat.reshape(_B, _K, _D), hidden, new_memory)


# speedup vs baseline: 86.4266x; 1.7077x over previous
import functools

import jax
import jax.numpy as jnp
from jax import lax
from jax.experimental import pallas as pl
from jax.experimental.pallas import tpu as pltpu
from jax.experimental.pallas import tpu_sc as plsc

_N = 100000
_D = 128
_R = 16
_K = 8
_B = 1024
_CBLK = 2048
_NBLK = 49
_NP = _CBLK * _NBLK
_G = _NBLK * 128
_GB = 896
_NGB = _G // _GB
_U = _B * _K
_RB = 256
_CJ = 128
_NJ = _U // _CJ
_BIG = 2**30


def _qmean(query):
    lq = query.shape[1]

    def body(q_ref, o_ref):
        acc = q_ref[:, 0, :]
        for l in range(1, lq):
            acc = acc + q_ref[:, l, :]
        o_ref[...] = acc * (1.0 / lq)

    return pl.pallas_call(
        body,
        grid=(4,),
        in_specs=[pl.BlockSpec((_B // 4, lq, _D), lambda g: (g, 0, 0))],
        out_specs=pl.BlockSpec((_B // 4, _D), lambda g: (g, 0)),
        out_shape=jax.ShapeDtypeStruct((_B, _D), jnp.float32),
    )(query)


def _score_groupmax(q, base, lora_a, lora_b):
    def body(q_ref, b_ref, la_ref, lb_ref, mem_ref, sc_ref, gm_ref, cp_ref):
        g = pl.program_id(0)
        base_blk = b_ref[...]
        cp_ref[...] = base_blk
        mem_blk = base_blk + jnp.dot(la_ref[...], lb_ref[...],
                                     preferred_element_type=jnp.float32)
        mem_ref[...] = mem_blk
        scores = lax.dot_general(q_ref[...], mem_blk, (((1,), (1,)), ((), ())),
                                 preferred_element_type=jnp.float32)
        col = lax.broadcasted_iota(jnp.int32, (_B, _CBLK), 1) + g * _CBLK
        scores = jnp.where(col < _N, scores, -jnp.inf)
        sc_ref[...] = scores
        m = scores[:, 0:128]
        for j in range(1, _CBLK // 128):
            m = jnp.maximum(m, scores[:, j * 128:(j + 1) * 128])
        gm_ref[...] = m

    return pl.pallas_call(
        body,
        grid=(_NBLK,),
        in_specs=[
            pl.BlockSpec((_B, _D), lambda g: (0, 0)),
            pl.BlockSpec((_CBLK, _D), lambda g: (g, 0)),
            pl.BlockSpec((_CBLK, _R), lambda g: (g, 0)),
            pl.BlockSpec((_R, _D), lambda g: (0, 0)),
        ],
        out_specs=[
            pl.BlockSpec((_CBLK, _D), lambda g: (g, 0)),
            pl.BlockSpec((_B, _CBLK), lambda g: (0, g)),
            pl.BlockSpec((_B, 128), lambda g: (0, g)),
            pl.BlockSpec((_CBLK, _D), lambda g: (g, 0)),
        ],
        out_shape=[
            jax.ShapeDtypeStruct((_N, _D), jnp.float32),
            jax.ShapeDtypeStruct((_B, _NP), jnp.float32),
            jax.ShapeDtypeStruct((_B, _G), jnp.float32),
            jax.ShapeDtypeStruct((_N, _D), jnp.float32),
        ],
    )(q, base, lora_a, lora_b)


def _topk_groups(gmax):
    def body(gm_ref, out_ref, bv_s, bi_s):
        g = pl.program_id(0)

        @pl.when(g == 0)
        def _():
            bv_s[...] = jnp.full((_B, _K), -jnp.inf, jnp.float32)
            bi_s[...] = jnp.full((_B, _K), _BIG, jnp.int32)

        s = gm_ref[...]
        gid = lax.broadcasted_iota(jnp.int32, (_B, _GB), 1) + g * _GB
        bv_list, bi_list = [], []
        for _ in range(_K):
            m = jnp.max(s, axis=1, keepdims=True)
            isel = jnp.min(jnp.where(s == m, gid, _BIG), axis=1, keepdims=True)
            bv_list.append(m)
            bi_list.append(isel)
            s = jnp.where(gid == isel, -jnp.inf, s)
        cv = jnp.concatenate([bv_s[...]] + bv_list, axis=1)
        ci = jnp.concatenate([bi_s[...]] + bi_list, axis=1)
        nv, ni = [], []
        for _ in range(_K):
            m = jnp.max(cv, axis=1, keepdims=True)
            isel = jnp.min(jnp.where(cv == m, ci, _BIG), axis=1, keepdims=True)
            nv.append(m)
            ni.append(isel)
            cv = jnp.where(ci == isel, -jnp.inf, cv)
        bv_s[...] = jnp.concatenate(nv, axis=1)
        bi_s[...] = jnp.concatenate(ni, axis=1)

        @pl.when(g == _NGB - 1)
        def _():
            out_ref[...] = bi_s[...]

    return pl.pallas_call(
        body,
        grid=(_NGB,),
        in_specs=[pl.BlockSpec((_B, _GB), lambda g: (0, g))],
        out_specs=pl.BlockSpec((_B, _K), lambda g: (0, 0)),
        out_shape=jax.ShapeDtypeStruct((_B, _K), jnp.int32),
        scratch_shapes=[pltpu.VMEM((_B, _K), jnp.float32),
                        pltpu.VMEM((_B, _K), jnp.int32)],
    )(gmax)


def _sc_gather_scores(tab, midx):
    mesh = plsc.VectorSubcoreMesh(core_axis_name="c", subcore_axis_name="s")
    rpw = _B // 32

    @functools.partial(
        pl.kernel,
        mesh=mesh,
        out_type=jax.ShapeDtypeStruct((_B, 128), jnp.float32),
        scratch_types=[pltpu.VMEM((rpw, 128), jnp.int32),
                       pltpu.VMEM((rpw, 128), jnp.float32),
                       pltpu.SemaphoreType.DMA],
    )
    def k(tab_hbm, idx_hbm, out_hbm, idx_v, buf_v, sem):
        wid = lax.axis_index("s") * 2 + lax.axis_index("c")
        pltpu.sync_copy(idx_hbm.at[pl.ds(wid * rpw, rpw)], idx_v)
        for half in range(2):
            cps = [pltpu.async_copy(tab_hbm.at[idx_v.at[half * 16 + r]],
                                    buf_v.at[half * 16 + r], sem)
                   for r in range(16)]
            for cp in cps:
                cp.wait()
        pltpu.sync_copy(buf_v, out_hbm.at[pl.ds(wid * rpw, rpw)])

    return k(tab, midx)


def _topk_members(ms, mcol):
    def body(ms_ref, mc_ref, out_ref):
        s = ms_ref[...]
        colv = mc_ref[...]
        idx_list = []
        for _ in range(_K):
            m = jnp.max(s, axis=1, keepdims=True)
            isel = jnp.min(jnp.where(s == m, colv, _BIG), axis=1, keepdims=True)
            idx_list.append(isel)
            s = jnp.where(colv == isel, -jnp.inf, s)
        out_ref[...] = jnp.concatenate(idx_list, axis=1)

    return pl.pallas_call(
        body,
        out_shape=jax.ShapeDtypeStruct((_B, _K), jnp.int32),
    )(ms, mcol)


def _sc_gather(mem, base, idx2d):
    mesh = plsc.VectorSubcoreMesh(core_axis_name="c", subcore_axis_name="s")
    nrows = idx2d.shape[0]
    rpw = nrows // 32
    ipw = rpw * 128

    @functools.partial(
        pl.kernel,
        mesh=mesh,
        out_type=(jax.ShapeDtypeStruct((_U, _D), jnp.float32),
                  jax.ShapeDtypeStruct((_U, _D), jnp.float32)),
        scratch_types=[pltpu.VMEM((rpw, 128), jnp.int32),
                       pltpu.VMEM((ipw, _D), jnp.float32),
                       pltpu.VMEM((ipw, _D), jnp.float32),
                       pltpu.SemaphoreType.DMA],
    )
    def k(mem_hbm, base_hbm, idx_hbm, ret_out, br_out, idx_v, rows_v, rows2_v, sem):
        wid = lax.axis_index("s") * 2 + lax.axis_index("c")
        pltpu.sync_copy(idx_hbm.at[pl.ds(wid * rpw, rpw)], idx_v)
        for c in range(rpw):
            pltpu.async_copy(mem_hbm.at[idx_v.at[c]],
                             rows_v.at[pl.ds(c * 128, 128)], sem).wait()
            pltpu.async_copy(base_hbm.at[idx_v.at[c]],
                             rows2_v.at[pl.ds(c * 128, 128)], sem).wait()
        pltpu.sync_copy(rows_v, ret_out.at[pl.ds(wid * ipw, ipw)])
        pltpu.sync_copy(rows2_v, br_out.at[pl.ds(wid * ipw, ipw)])

    return k(mem, base, idx2d)


def _gru(ret3, w_ih, b_ih2, b_hh2, w_w, w_b2):
    def body(r_ref, wih_ref, bih_ref, bhh_ref, ww_ref, wb_ref,
             h_ref, w_ref, sp_ref):
        x = r_ref[:, 0, :]
        for kk in range(1, _K):
            x = x + r_ref[:, kk, :]
        gi = lax.dot_general(x, wih_ref[...], (((1,), (1,)), ((), ())),
                             preferred_element_type=jnp.float32) + bih_ref[...]
        gh = bhh_ref[...]
        i_r = gi[:, :_D]
        i_z = gi[:, _D:2 * _D]
        i_n = gi[:, 2 * _D:]
        h_r = gh[:, :_D]
        h_z = gh[:, _D:2 * _D]
        h_n = gh[:, 2 * _D:]
        r = jax.nn.sigmoid(i_r + h_r)
        z = jax.nn.sigmoid(i_z + h_z)
        n = jnp.tanh(i_n + r * h_n)
        hidden = (1.0 - z) * n
        h_ref[...] = hidden
        wl = jnp.sum(hidden * ww_ref[...], axis=1, keepdims=True) + wb_ref[0, 0]
        w_ref[...] = jnp.broadcast_to(jax.nn.sigmoid(wl), (_B, _D))
        sp = jnp.maximum(wl, 0.0) + jnp.log1p(jnp.exp(-jnp.abs(wl)))
        sp_ref[...] = jnp.broadcast_to(sp, (_B, _D))

    return pl.pallas_call(
        body,
        in_specs=[
            pl.BlockSpec(memory_space=pltpu.VMEM),
            pl.BlockSpec(memory_space=pltpu.VMEM),
            pl.BlockSpec(memory_space=pltpu.VMEM),
            pl.BlockSpec(memory_space=pltpu.VMEM),
            pl.BlockSpec(memory_space=pltpu.VMEM),
            pl.BlockSpec(memory_space=pltpu.SMEM),
        ],
        out_shape=[
            jax.ShapeDtypeStruct((_B, _D), jnp.float32),
            jax.ShapeDtypeStruct((_B, _D), jnp.float32),
            jax.ShapeDtypeStruct((_B, _D), jnp.float32),
        ],
    )(ret3, w_ih, b_ih2, b_hh2, w_w, w_b2)


def _write_sums(idxr_w, idxc, spc):
    def body(idxr_ref, idxc_ref, spc_ref, tot_ref, lat_ref):
        g = pl.program_id(0)
        ir = idxr_ref[...]
        tot = jnp.zeros((_RB, 1), jnp.float32)
        lat = jnp.zeros((_RB, 1), jnp.float32)
        rowg = lax.broadcasted_iota(jnp.int32, (_RB, _CJ), 0) + g * _RB
        for c in range(_NJ):
            ic = idxc_ref[:, c * _CJ:(c + 1) * _CJ]
            spv = spc_ref[:, c * _CJ:(c + 1) * _CJ]
            eq = ir == ic
            tot = tot + jnp.sum(jnp.where(eq, spv, 0.0), axis=1, keepdims=True)
            colg = lax.broadcasted_iota(jnp.int32, (_RB, _CJ), 1) + c * _CJ
            lat = lat + jnp.sum(jnp.where(eq & (colg > rowg), spv, 0.0),
                                axis=1, keepdims=True)
        tot_ref[...] = jnp.broadcast_to(tot, (_RB, _D))
        lat_ref[...] = jnp.broadcast_to(lat, (_RB, _D))

    return pl.pallas_call(
        body,
        grid=(_U // _RB,),
        in_specs=[
            pl.BlockSpec((_RB, _D), lambda g: (g, 0)),
            pl.BlockSpec((1, _U), lambda g: (0, 0)),
            pl.BlockSpec((1, _U), lambda g: (0, 0)),
        ],
        out_specs=[
            pl.BlockSpec((_RB, _D), lambda g: (g, 0)),
            pl.BlockSpec((_RB, _D), lambda g: (g, 0)),
        ],
        out_shape=[
            jax.ShapeDtypeStruct((_U, _D), jnp.float32),
            jax.ShapeDtypeStruct((_U, _D), jnp.float32),
        ],
    )(idxr_w, idxc, spc)


def _write_rows(idxr_w, idxc, latc, wc, v_u, base_rows, tot_w):
    def body(idxr_ref, idxc_ref, latc_ref, wc_ref, v_ref, br_ref, tot_ref,
             out_ref):
        ir = idxr_ref[...]
        coef = wc_ref[...] * jnp.exp(-latc_ref[...])
        acc = jnp.zeros((_RB, _D), jnp.float32)
        for c in range(_NJ):
            ic = idxc_ref[:, c * _CJ:(c + 1) * _CJ]
            m = jnp.where(ir == ic, coef[:, c * _CJ:(c + 1) * _CJ], 0.0)
            acc = acc + jnp.dot(m, v_ref[c * _CJ:(c + 1) * _CJ, :],
                                preferred_element_type=jnp.float32)
        out_ref[...] = jnp.exp(-tot_ref[...]) * br_ref[...] + acc

    return pl.pallas_call(
        body,
        grid=(_U // _RB,),
        in_specs=[
            pl.BlockSpec((_RB, _D), lambda g: (g, 0)),
            pl.BlockSpec((1, _U), lambda g: (0, 0)),
            pl.BlockSpec((1, _U), lambda g: (0, 0)),
            pl.BlockSpec((1, _U), lambda g: (0, 0)),
            pl.BlockSpec((_U, _D), lambda g: (0, 0)),
            pl.BlockSpec((_RB, _D), lambda g: (g, 0)),
            pl.BlockSpec((_RB, _D), lambda g: (g, 0)),
        ],
        out_specs=pl.BlockSpec((_RB, _D), lambda g: (g, 0)),
        out_shape=jax.ShapeDtypeStruct((_U, _D), jnp.float32),
    )(idxr_w, idxc, latc, wc, v_u, base_rows, tot_w)


def _sc_scatter(newmem, idx2d, rows):
    mesh = plsc.VectorSubcoreMesh(core_axis_name="c", subcore_axis_name="s")
    nrows = idx2d.shape[0]
    rpw = nrows // 32
    ipw = rpw * 128

    @functools.partial(
        pl.kernel,
        mesh=mesh,
        out_type=(),
        scratch_types=[pltpu.VMEM((rpw, 128), jnp.int32),
                       pltpu.VMEM((ipw, _D), jnp.float32),
                       pltpu.SemaphoreType.DMA],
    )
    def k(idx_hbm, rows_hbm, out_hbm, idx_v, rows_v, sem):
        wid = lax.axis_index("s") * 2 + lax.axis_index("c")
        pltpu.sync_copy(idx_hbm.at[pl.ds(wid * rpw, rpw)], idx_v)
        pltpu.sync_copy(rows_hbm.at[pl.ds(wid * ipw, ipw)], rows_v)
        for c in range(rpw):
            pltpu.async_copy(rows_v.at[pl.ds(c * 128, 128)],
                             out_hbm.at[idx_v.at[c]], sem).wait()

    out_ref = jax.new_ref(newmem)
    k(idx2d, rows, out_ref)
    return out_ref[...]


def kernel(query, base_memory, lora_A, lora_B, gru_w_ih, gru_w_hh, gru_b_ih,
           gru_b_hh, write_w, write_b, erase_w, erase_b):
    q = _qmean(query)
    mem, scores, gmax, newmem = _score_groupmax(q, base_memory, lora_A, lora_B)
    top_gid = _topk_groups(gmax)
    blk = top_gid // 128
    lane = top_gid - blk * 128
    j16 = jnp.arange(16, dtype=jnp.int32)
    mcol = (blk * _CBLK + lane)[:, :, None] + 128 * j16
    mcol = mcol.reshape(_B, 128)
    flat = (jnp.arange(_B, dtype=jnp.int32) * _NP)[:, None] + mcol
    ms = _sc_gather_scores(scores.reshape(-1), flat)
    top_idx = _topk_members(ms, mcol)
    idx2d = top_idx.reshape(_U // 128, 128)
    retrieved_flat, base_rows = _sc_gather(mem, base_memory, idx2d)
    hidden, w128, sp128 = _gru(retrieved_flat.reshape(_B, _K, _D), gru_w_ih,
                               gru_b_ih.reshape(1, -1), gru_b_hh.reshape(1, -1),
                               write_w, write_b.reshape(1, 1))
    w = w128[:, :1]
    sp = sp128[:, :1]
    idxf = top_idx.reshape(-1).astype(jnp.float32)
    idxr_w = jnp.broadcast_to(idxf[:, None], (_U, _D))
    idxc = idxf.reshape(1, _U)
    spc = jnp.broadcast_to(sp, (_B, _K)).reshape(1, _U)
    wc = jnp.broadcast_to(w, (_B, _K)).reshape(1, _U)
    tot_w, lat_w = _write_sums(idxr_w, idxc, spc)
    v_u = jnp.broadcast_to(q[:, None, :], (_B, _K, _D)).reshape(_U, _D)
    rows = _write_rows(idxr_w, idxc, lat_w[:, :1].reshape(1, _U), wc, v_u,
                       base_rows, tot_w)
    new_memory = _sc_scatter(newmem, idx2d, rows)
    return (retrieved_flat.reshape(_B, _K, _D), hidden, new_memory)


# speedup vs baseline: 96.8706x; 1.1208x over previous
import functools

import jax
import jax.numpy as jnp
from jax import lax
from jax.experimental import pallas as pl
from jax.experimental.pallas import tpu as pltpu
from jax.experimental.pallas import tpu_sc as plsc

_N = 100000
_D = 128
_R = 16
_K = 8
_B = 1024
_CBLK = 2048
_NBLK = 49
_NP = _CBLK * _NBLK
_G = _NBLK * 128
_GB = 896
_NGB = _G // _GB
_U = _B * _K
_RB = 256
_CJ = 128
_NJ = _U // _CJ
_BIG = 2**30


def _qmean(query):
    lq = query.shape[1]

    def body(q_ref, o_ref):
        acc = q_ref[:, 0, :]
        for l in range(1, lq):
            acc = acc + q_ref[:, l, :]
        o_ref[...] = acc * (1.0 / lq)

    return pl.pallas_call(
        body,
        grid=(4,),
        in_specs=[pl.BlockSpec((_B // 4, lq, _D), lambda g: (g, 0, 0))],
        out_specs=pl.BlockSpec((_B // 4, _D), lambda g: (g, 0)),
        out_shape=jax.ShapeDtypeStruct((_B, _D), jnp.float32),
    )(query)


def _score_groupmax(q, base, lora_a, lora_b):
    def body(q_ref, b_ref, la_ref, lb_ref, sc_ref, gm_ref, cp_ref):
        g = pl.program_id(0)
        base_blk = b_ref[...]
        cp_ref[...] = base_blk
        mem_blk = base_blk + jnp.dot(la_ref[...], lb_ref[...],
                                     preferred_element_type=jnp.float32)
        scores = lax.dot_general(q_ref[...], mem_blk, (((1,), (1,)), ((), ())),
                                 preferred_element_type=jnp.float32)
        col = lax.broadcasted_iota(jnp.int32, (_B, _CBLK), 1) + g * _CBLK
        scores = jnp.where(col < _N, scores, -jnp.inf)
        sc_ref[...] = scores
        m = scores[:, 0:128]
        for j in range(1, _CBLK // 128):
            m = jnp.maximum(m, scores[:, j * 128:(j + 1) * 128])
        gm_ref[...] = m

    return pl.pallas_call(
        body,
        grid=(_NBLK,),
        in_specs=[
            pl.BlockSpec((_B, _D), lambda g: (0, 0)),
            pl.BlockSpec((_CBLK, _D), lambda g: (g, 0)),
            pl.BlockSpec((_CBLK, _R), lambda g: (g, 0)),
            pl.BlockSpec((_R, _D), lambda g: (0, 0)),
        ],
        out_specs=[
            pl.BlockSpec((_B, _CBLK), lambda g: (0, g)),
            pl.BlockSpec((_B, 128), lambda g: (0, g)),
            pl.BlockSpec((_CBLK, _D), lambda g: (g, 0)),
        ],
        out_shape=[
            jax.ShapeDtypeStruct((_B, _NP), jnp.float32),
            jax.ShapeDtypeStruct((_B, _G), jnp.float32),
            jax.ShapeDtypeStruct((_N, _D), jnp.float32),
        ],
    )(q, base, lora_a, lora_b)


def _topk_groups(gmax):
    def body(gm_ref, out_ref, bv_s, bi_s):
        g = pl.program_id(0)

        @pl.when(g == 0)
        def _():
            bv_s[...] = jnp.full((_B, _K), -jnp.inf, jnp.float32)
            bi_s[...] = jnp.full((_B, _K), _BIG, jnp.int32)

        s = gm_ref[...]
        gid = lax.broadcasted_iota(jnp.int32, (_B, _GB), 1) + g * _GB
        bv_list, bi_list = [], []
        for _ in range(_K):
            m = jnp.max(s, axis=1, keepdims=True)
            isel = jnp.min(jnp.where(s == m, gid, _BIG), axis=1, keepdims=True)
            bv_list.append(m)
            bi_list.append(isel)
            s = jnp.where(gid == isel, -jnp.inf, s)
        cv = jnp.concatenate([bv_s[...]] + bv_list, axis=1)
        ci = jnp.concatenate([bi_s[...]] + bi_list, axis=1)
        nv, ni = [], []
        for _ in range(_K):
            m = jnp.max(cv, axis=1, keepdims=True)
            isel = jnp.min(jnp.where(cv == m, ci, _BIG), axis=1, keepdims=True)
            nv.append(m)
            ni.append(isel)
            cv = jnp.where(ci == isel, -jnp.inf, cv)
        bv_s[...] = jnp.concatenate(nv, axis=1)
        bi_s[...] = jnp.concatenate(ni, axis=1)

        @pl.when(g == _NGB - 1)
        def _():
            out_ref[...] = bi_s[...]

    return pl.pallas_call(
        body,
        grid=(_NGB,),
        in_specs=[pl.BlockSpec((_B, _GB), lambda g: (0, g))],
        out_specs=pl.BlockSpec((_B, _K), lambda g: (0, 0)),
        out_shape=jax.ShapeDtypeStruct((_B, _K), jnp.int32),
        scratch_shapes=[pltpu.VMEM((_B, _K), jnp.float32),
                        pltpu.VMEM((_B, _K), jnp.int32)],
    )(gmax)


def _sc_gather_scores(tab, midx):
    mesh = plsc.VectorSubcoreMesh(core_axis_name="c", subcore_axis_name="s")
    rpw = _B // 32

    @functools.partial(
        pl.kernel,
        mesh=mesh,
        out_type=jax.ShapeDtypeStruct((_B, 128), jnp.float32),
        scratch_types=[pltpu.VMEM((rpw, 128), jnp.int32),
                       pltpu.VMEM((rpw, 128), jnp.float32),
                       pltpu.SemaphoreType.DMA],
    )
    def k(tab_hbm, idx_hbm, out_hbm, idx_v, buf_v, sem):
        wid = lax.axis_index("s") * 2 + lax.axis_index("c")
        pltpu.sync_copy(idx_hbm.at[pl.ds(wid * rpw, rpw)], idx_v)
        for half in range(2):
            cps = [pltpu.async_copy(tab_hbm.at[idx_v.at[half * 16 + r]],
                                    buf_v.at[half * 16 + r], sem)
                   for r in range(16)]
            for cp in cps:
                cp.wait()
        pltpu.sync_copy(buf_v, out_hbm.at[pl.ds(wid * rpw, rpw)])

    return k(tab, midx)


def _topk_members(ms, mcol):
    def body(ms_ref, mc_ref, out_ref):
        s = ms_ref[...]
        colv = mc_ref[...]
        idx_list = []
        for _ in range(_K):
            m = jnp.max(s, axis=1, keepdims=True)
            isel = jnp.min(jnp.where(s == m, colv, _BIG), axis=1, keepdims=True)
            idx_list.append(isel)
            s = jnp.where(colv == isel, -jnp.inf, s)
        out_ref[...] = jnp.concatenate(idx_list, axis=1)

    return pl.pallas_call(
        body,
        out_shape=jax.ShapeDtypeStruct((_B, _K), jnp.int32),
    )(ms, mcol)


def _sc_gather(base, idx2d):
    mesh = plsc.VectorSubcoreMesh(core_axis_name="c", subcore_axis_name="s")
    nrows = idx2d.shape[0]
    rpw = nrows // 32
    ipw = rpw * 128

    @functools.partial(
        pl.kernel,
        mesh=mesh,
        out_type=jax.ShapeDtypeStruct((_U, _D), jnp.float32),
        scratch_types=[pltpu.VMEM((rpw, 128), jnp.int32),
                       pltpu.VMEM((ipw, _D), jnp.float32),
                       pltpu.SemaphoreType.DMA],
    )
    def k(base_hbm, idx_hbm, br_out, idx_v, rows_v, sem):
        wid = lax.axis_index("s") * 2 + lax.axis_index("c")
        pltpu.sync_copy(idx_hbm.at[pl.ds(wid * rpw, rpw)], idx_v)
        for c in range(rpw):
            pltpu.async_copy(base_hbm.at[idx_v.at[c]],
                             rows_v.at[pl.ds(c * 128, 128)], sem).wait()
        pltpu.sync_copy(rows_v, br_out.at[pl.ds(wid * ipw, ipw)])

    return k(base, idx2d)


def _gru(br3, la3, lora_b, w_ih, b_ih2, b_hh2, w_w, w_b2):
    def body(br_ref, la_ref, lb_ref, wih_ref, bih_ref, bhh_ref, ww_ref, wb_ref,
             ret_ref, h_ref, w_ref, sp_ref):
        xb = br_ref[:, 0, :]
        xl = la_ref[:, 0, :]
        for kk in range(1, _K):
            xb = xb + br_ref[:, kk, :]
            xl = xl + la_ref[:, kk, :]
        lb = lb_ref[...]
        for kk in range(_K):
            ret_ref[:, kk, :] = br_ref[:, kk, :] + jnp.dot(
                la_ref[:, kk, :], lb, preferred_element_type=jnp.float32)
        x = xb + jnp.dot(xl, lb, preferred_element_type=jnp.float32)
        gi = lax.dot_general(x, wih_ref[...], (((1,), (1,)), ((), ())),
                             preferred_element_type=jnp.float32) + bih_ref[...]
        gh = bhh_ref[...]
        i_r = gi[:, :_D]
        i_z = gi[:, _D:2 * _D]
        i_n = gi[:, 2 * _D:]
        h_r = gh[:, :_D]
        h_z = gh[:, _D:2 * _D]
        h_n = gh[:, 2 * _D:]
        r = jax.nn.sigmoid(i_r + h_r)
        z = jax.nn.sigmoid(i_z + h_z)
        n = jnp.tanh(i_n + r * h_n)
        hidden = (1.0 - z) * n
        h_ref[...] = hidden
        wl = jnp.sum(hidden * ww_ref[...], axis=1, keepdims=True) + wb_ref[0, 0]
        w_ref[...] = jnp.broadcast_to(jax.nn.sigmoid(wl), (_B, _D))
        sp = jnp.maximum(wl, 0.0) + jnp.log1p(jnp.exp(-jnp.abs(wl)))
        sp_ref[...] = jnp.broadcast_to(sp, (_B, _D))

    return pl.pallas_call(
        body,
        in_specs=[
            pl.BlockSpec(memory_space=pltpu.VMEM),
            pl.BlockSpec(memory_space=pltpu.VMEM),
            pl.BlockSpec(memory_space=pltpu.VMEM),
            pl.BlockSpec(memory_space=pltpu.VMEM),
            pl.BlockSpec(memory_space=pltpu.VMEM),
            pl.BlockSpec(memory_space=pltpu.VMEM),
            pl.BlockSpec(memory_space=pltpu.VMEM),
            pl.BlockSpec(memory_space=pltpu.SMEM),
        ],
        out_shape=[
            jax.ShapeDtypeStruct((_B, _K, _D), jnp.float32),
            jax.ShapeDtypeStruct((_B, _D), jnp.float32),
            jax.ShapeDtypeStruct((_B, _D), jnp.float32),
            jax.ShapeDtypeStruct((_B, _D), jnp.float32),
        ],
    )(br3, la3, lora_b, w_ih, b_ih2, b_hh2, w_w, w_b2)


def _write_sums(idxr_w, tk_cols, sp_cols):
    def body(idxr_ref, tk_ref, spb_ref, tot_ref, lat_ref):
        g = pl.program_id(0)
        ir = idxr_ref[...]
        rowb = (lax.broadcasted_iota(jnp.int32, (_RB, 128), 0)
                + g * _RB) // _K
        tot = jnp.zeros((_RB, 1), jnp.float32)
        lat = jnp.zeros((_RB, 1), jnp.float32)
        for c in range(_B // 128):
            lo, hi = c * 128, (c + 1) * 128
            mb = ir == tk_ref[0:1, lo:hi]
            for kk in range(1, _K):
                mb = mb | (ir == tk_ref[kk:kk + 1, lo:hi])
            spb = spb_ref[:, lo:hi]
            tot = tot + jnp.sum(jnp.where(mb, spb, 0.0), axis=1, keepdims=True)
            colb = lax.broadcasted_iota(jnp.int32, (_RB, 128), 1) + c * 128
            lat = lat + jnp.sum(jnp.where(mb & (colb > rowb), spb, 0.0),
                                axis=1, keepdims=True)
        tot_ref[...] = jnp.broadcast_to(tot, (_RB, _D))
        lat_ref[...] = jnp.broadcast_to(lat, (_RB, _D))

    return pl.pallas_call(
        body,
        grid=(_U // _RB,),
        in_specs=[
            pl.BlockSpec((_RB, _D), lambda g: (g, 0)),
            pl.BlockSpec((_K, _B), lambda g: (0, 0)),
            pl.BlockSpec((1, _B), lambda g: (0, 0)),
        ],
        out_specs=[
            pl.BlockSpec((_RB, _D), lambda g: (g, 0)),
            pl.BlockSpec((_RB, _D), lambda g: (g, 0)),
        ],
        out_shape=[
            jax.ShapeDtypeStruct((_U, _D), jnp.float32),
            jax.ShapeDtypeStruct((_U, _D), jnp.float32),
        ],
    )(idxr_w, tk_cols, sp_cols)


def _write_rows(idxr_w, tk_cols, lat_cols, w_cols, q, base_rows, tot_w):
    def body(idxr_ref, tk_ref, lat_ref, wb_ref, q_ref, br_ref, tot_ref,
             out_ref):
        ir = idxr_ref[...]
        acc = jnp.zeros((_RB, _D), jnp.float32)
        for c in range(_B // 128):
            lo, hi = c * 128, (c + 1) * 128
            wb = wb_ref[:, lo:hi]
            mcoef = jnp.where(ir == tk_ref[0:1, lo:hi],
                              wb * jnp.exp(-lat_ref[0:1, lo:hi]), 0.0)
            for kk in range(1, _K):
                mcoef = mcoef + jnp.where(
                    ir == tk_ref[kk:kk + 1, lo:hi],
                    wb * jnp.exp(-lat_ref[kk:kk + 1, lo:hi]), 0.0)
            acc = acc + jnp.dot(mcoef, q_ref[lo:hi, :],
                                preferred_element_type=jnp.float32)
        out_ref[...] = jnp.exp(-tot_ref[...]) * br_ref[...] + acc

    return pl.pallas_call(
        body,
        grid=(_U // _RB,),
        in_specs=[
            pl.BlockSpec((_RB, _D), lambda g: (g, 0)),
            pl.BlockSpec((_K, _B), lambda g: (0, 0)),
            pl.BlockSpec((_K, _B), lambda g: (0, 0)),
            pl.BlockSpec((1, _B), lambda g: (0, 0)),
            pl.BlockSpec((_B, _D), lambda g: (0, 0)),
            pl.BlockSpec((_RB, _D), lambda g: (g, 0)),
            pl.BlockSpec((_RB, _D), lambda g: (g, 0)),
        ],
        out_specs=pl.BlockSpec((_RB, _D), lambda g: (g, 0)),
        out_shape=jax.ShapeDtypeStruct((_U, _D), jnp.float32),
    )(idxr_w, tk_cols, lat_cols, w_cols, q, base_rows, tot_w)


def _sc_scatter(newmem, idx2d, rows):
    mesh = plsc.VectorSubcoreMesh(core_axis_name="c", subcore_axis_name="s")
    nrows = idx2d.shape[0]
    rpw = nrows // 32
    ipw = rpw * 128

    @functools.partial(
        pl.kernel,
        mesh=mesh,
        out_type=(),
        scratch_types=[pltpu.VMEM((rpw, 128), jnp.int32),
                       pltpu.VMEM((ipw, _D), jnp.float32),
                       pltpu.SemaphoreType.DMA],
    )
    def k(idx_hbm, rows_hbm, out_hbm, idx_v, rows_v, sem):
        wid = lax.axis_index("s") * 2 + lax.axis_index("c")
        pltpu.sync_copy(idx_hbm.at[pl.ds(wid * rpw, rpw)], idx_v)
        pltpu.sync_copy(rows_hbm.at[pl.ds(wid * ipw, ipw)], rows_v)
        for c in range(rpw):
            pltpu.async_copy(rows_v.at[pl.ds(c * 128, 128)],
                             out_hbm.at[idx_v.at[c]], sem).wait()

    out_ref = jax.new_ref(newmem)
    k(idx2d, rows, out_ref)
    return out_ref[...]


def kernel(query, base_memory, lora_A, lora_B, gru_w_ih, gru_w_hh, gru_b_ih,
           gru_b_hh, write_w, write_b, erase_w, erase_b):
    q = _qmean(query)
    scores, gmax, newmem = _score_groupmax(q, base_memory, lora_A, lora_B)
    top_gid = _topk_groups(gmax)
    blk = top_gid // 128
    lane = top_gid - blk * 128
    j16 = jnp.arange(16, dtype=jnp.int32)
    mcol = (blk * _CBLK + lane)[:, :, None] + 128 * j16
    mcol = mcol.reshape(_B, 128)
    flat = (jnp.arange(_B, dtype=jnp.int32) * _NP)[:, None] + mcol
    ms = _sc_gather_scores(scores.reshape(-1), flat)
    top_idx = _topk_members(ms, mcol)
    idx2d = top_idx.reshape(_U // 128, 128)
    base_rows = _sc_gather(base_memory, idx2d)
    la_idx = ((top_idx.reshape(-1) * _R)[:, None]
              + jnp.arange(_R, dtype=jnp.int32)).reshape(_B, 128)
    la_g = _sc_gather_scores(lora_A.reshape(-1), la_idx).reshape(_U, _R)
    ret3, hidden, w128, sp128 = _gru(base_rows.reshape(_B, _K, _D),
                                     la_g.reshape(_B, _K, _R), lora_B,
                                     gru_w_ih, gru_b_ih.reshape(1, -1),
                                     gru_b_hh.reshape(1, -1),
                                     write_w, write_b.reshape(1, 1))
    idxf = top_idx.reshape(-1).astype(jnp.float32)
    idxr_w = jnp.broadcast_to(idxf[:, None], (_U, _D))
    tk_cols = top_idx.astype(jnp.float32).T
    sp_cols = sp128[:, 0].reshape(1, _B)
    w_cols = w128[:, 0].reshape(1, _B)
    tot_w, lat_w = _write_sums(idxr_w, tk_cols, sp_cols)
    lat_cols = lat_w[:, 0].reshape(_B, _K).T
    rows = _write_rows(idxr_w, tk_cols, lat_cols, w_cols, q, base_rows, tot_w)
    new_memory = _sc_scatter(newmem, idx2d, rows)
    return (ret3, hidden, new_memory)
